# baseline (speedup 1.0000x reference)
import jax
import jax.numpy as jnp
from jax import lax
from jax.experimental import pallas as pl
from jax.experimental.pallas import tpu as pltpu
from jax.experimental.pallas import tpu_sc as plsc

_EMBED = 64
_SCALE = 8.0
_LANES = 16
_BBLK = 128
_EH = _EMBED // 8


def kernel(inputTensor, table):
    batch, seq = inputTensor.shape
    num_idx = batch * seq

    info = plsc.get_sparse_core_info()
    n_workers = info.num_cores * info.num_subcores
    idx_per_worker = _BBLK * seq
    n_tiles = seq * _EH * (batch // _BBLK)

    idx_t = inputTensor.T.reshape(num_idx)

    mesh = plsc.VectorSubcoreMesh(
        core_axis_name="core", subcore_axis_name="subcore"
    )

    @jax.jit
    @pl.kernel(
        out_type=jax.ShapeDtypeStruct((n_tiles, 8, _BBLK), table.dtype),
        mesh=mesh,
        scratch_types=[
            pltpu.VMEM((2, _BBLK), jnp.int32),
            pltpu.VMEM((2, _BBLK, _EMBED), jnp.float32),
            pltpu.VMEM((2, _EH, 8, _BBLK), jnp.float32),
            pltpu.SemaphoreType.DMA((2,)),
            pltpu.SemaphoreType.DMA((2,)),
            pltpu.SemaphoreType.DMA((2,)),
        ],
        compiler_params=pltpu.CompilerParams(
            use_tc_tiling_on_sc=False, needs_layout_passes=False
        ),
    )
    def gather_scale(table_hbm, idx_hbm, out_hbm, givec, rows, tbuf, gsem, osem, isem):
        w = lax.axis_index("subcore") * info.num_cores + lax.axis_index("core")
        iota16 = lax.iota(jnp.int32, 16)

        def load_gidx_sync(s, k):
            pltpu.sync_copy(
                idx_hbm.at[pl.ds(s * batch + w * _BBLK, _BBLK)], givec.at[k]
            )

        def start_gidx(s, k):
            pltpu.async_copy(
                idx_hbm.at[pl.ds(s * batch + w * _BBLK, _BBLK)],
                givec.at[k],
                isem.at[k],
            )

        def wait_gidx(k):
            pltpu.make_async_copy(
                idx_hbm.at[pl.ds(0, _BBLK)], givec.at[k], isem.at[k]
            ).wait()

        def start_gather(k):
            pltpu.async_copy(
                table_hbm.at[givec.at[k]], rows.at[k], gsem.at[k]
            )

        def wait_gather(k):
            pltpu.make_async_copy(
                table_hbm.at[pl.ds(0, _BBLK)], rows.at[k], gsem.at[k]
            ).wait()

        def transpose_scale(k):
            src = rows.at[k]
            dst = tbuf.at[k]

            def eh_body(eh, carry):
                for el in range(8):
                    col = jnp.broadcast_to(eh * 8 + el, (_LANES,)).astype(jnp.int32)
                    for g in range(_BBLK // _LANES):
                        row_idx = iota16 + g * _LANES
                        vals = plsc.load_gather(src, [row_idx, col])
                        dst[eh, el, pl.ds(g * _LANES, _LANES)] = vals * _SCALE
                return carry

            lax.fori_loop(0, _EH, eh_body, 0)

        def start_out(s, k):
            for eh in range(_EH):
                pltpu.async_copy(
                    tbuf.at[k].at[eh],
                    out_hbm.at[(s * _EH + eh) * n_workers + w],
                    osem.at[k],
                )

        def wait_out(k):
            pltpu.make_async_copy(
                tbuf.at[k], out_hbm.at[pl.ds(0, _EH)], osem.at[k]
            ).wait()

        load_gidx_sync(0, 0)
        load_gidx_sync(1, 1)
        start_gather(0)

        def turn(j, carry):
            for k in range(2):
                s = j * 2 + k
                k2 = (k + 1) % 2

                @pl.when(s + 1 < seq)
                def _prefetch(s=s, k2=k2):
                    @pl.when(s >= 1)
                    def _idx_ready(k2=k2):
                        wait_gidx(k2)

                    start_gather(k2)

                wait_gather(k)

                @pl.when(s + 2 < seq)
                def _prefetch_idx(s=s, k=k):
                    start_gidx(s + 2, k)

                @pl.when(s >= 2)
                def _free_tbuf(k=k):
                    wait_out(k)

                transpose_scale(k)
                start_out(s, k)
            return carry

        lax.fori_loop(0, seq // 2, turn, 0)

        for k in range(2):
            wait_out(k)

    out3 = gather_scale(table, idx_t)
    t5 = out3.reshape(seq, _EH, batch // _BBLK, 8, _BBLK)
    return t5.transpose(2, 4, 0, 1, 3).reshape(batch, seq, _EMBED)


# speedup vs baseline: 1.5905x; 1.5905x over previous
import jax
import jax.numpy as jnp
from jax import lax
from jax.experimental import pallas as pl
from jax.experimental.pallas import tpu as pltpu
from jax.experimental.pallas import tpu_sc as plsc

_EMBED = 64
_SCALE = 8.0
_NBUF = 4
_LANES = 16
_SPLIT = 104


def kernel(inputTensor, table):
    batch, seq = inputTensor.shape
    num_idx = batch * seq
    idx = inputTensor.reshape(num_idx)

    info = plsc.get_sparse_core_info()
    n_workers = info.num_cores * info.num_subcores
    b_per_worker = batch // n_workers
    idx_per_worker = b_per_worker * seq

    mesh = plsc.VectorSubcoreMesh(
        core_axis_name="core", subcore_axis_name="subcore"
    )

    @jax.jit
    @pl.kernel(
        out_type=jax.ShapeDtypeStruct((batch, seq, _EMBED), table.dtype),
        mesh=mesh,
        scratch_types=[
            pltpu.VMEM((idx_per_worker,), jnp.int32),
            pltpu.VMEM((_NBUF, seq, _EMBED), jnp.float32),
            pltpu.SemaphoreType.DMA((_NBUF,)),
            pltpu.SemaphoreType.DMA((_NBUF,)),
        ],
        compiler_params=pltpu.CompilerParams(use_tc_tiling_on_sc=False),
    )
    def gather_scale(table_hbm, idx_hbm, out_hbm, idx_v, buf, gsem, osem):
        wid = lax.axis_index("subcore") * info.num_cores + lax.axis_index("core")
        b0 = wid * b_per_worker
        pltpu.sync_copy(idx_hbm.at[pl.ds(b0 * seq, idx_per_worker)], idx_v)

        def start_gather(t, k):
            off = t * seq
            pltpu.async_copy(
                table_hbm.at[idx_v.at[pl.ds(off, _SPLIT)]],
                buf.at[k].at[pl.ds(0, _SPLIT)],
                gsem.at[k],
            )
            pltpu.async_copy(
                table_hbm.at[idx_v.at[pl.ds(off + _SPLIT, seq - _SPLIT)]],
                buf.at[k].at[pl.ds(_SPLIT, seq - _SPLIT)],
                gsem.at[k],
            )

        def wait_gather(k):
            pltpu.make_async_copy(
                table_hbm.at[pl.ds(0, seq)], buf.at[k], gsem.at[k]
            ).wait()

        def start_out(t, k):
            pltpu.async_copy(buf.at[k], out_hbm.at[b0 + t], osem.at[k])

        def wait_out(k):
            pltpu.make_async_copy(buf.at[k], out_hbm.at[b0], osem.at[k]).wait()

        def scale(k):
            dst = buf.at[k]

            def row(r, carry):
                for c in range(_EMBED // _LANES):
                    sl = pl.ds(c * _LANES, _LANES)
                    dst[r, sl] = dst[r, sl] * _SCALE
                return carry

            lax.fori_loop(0, seq, row, 0)

        for t in range(2):
            start_gather(t, t)

        def turn(j, carry):
            for k in range(_NBUF):
                t = j * _NBUF + k
                k2 = (k + 2) % _NBUF

                wait_gather(k)

                @pl.when(t + 2 < b_per_worker)
                def _start_ahead(t=t, k2=k2):
                    @pl.when(t >= 2)
                    def _free_slot(k2=k2):
                        wait_out(k2)

                    start_gather(t + 2, k2)

                scale(k)
                start_out(t, k)
            return carry

        lax.fori_loop(0, b_per_worker // _NBUF, turn, 0)

        for k in range(_NBUF):
            wait_out(k)

    out = gather_scale(table, idx)
    return out


# speedup vs baseline: 1.7371x; 1.0922x over previous
import jax
import jax.numpy as jnp
from jax import lax
from jax.experimental import pallas as pl
from jax.experimental.pallas import tpu as pltpu
from jax.experimental.pallas import tpu_sc as plsc

_EMBED = 64
_SCALE = 8.0
_LANES = 16
_BBLK = 128
_EH = _EMBED // 8


def kernel(inputTensor, table):
    batch, seq = inputTensor.shape
    num_idx = batch * seq

    info = plsc.get_sparse_core_info()
    n_workers = info.num_cores * info.num_subcores
    idx_per_worker = _BBLK * seq
    n_tiles = seq * _EH * (batch // _BBLK)

    idx_t = inputTensor.T.reshape(num_idx)

    mesh = plsc.VectorSubcoreMesh(
        core_axis_name="core", subcore_axis_name="subcore"
    )

    @jax.jit
    @pl.kernel(
        out_type=jax.ShapeDtypeStruct((n_tiles, 8, _BBLK), table.dtype),
        mesh=mesh,
        scratch_types=[
            pltpu.VMEM((2, _BBLK), jnp.int32),
            pltpu.VMEM((2, _BBLK, _EMBED), jnp.float32),
            pltpu.VMEM((2, _EMBED, _BBLK), jnp.float32),
            pltpu.SemaphoreType.DMA((2,)),
            pltpu.SemaphoreType.DMA((2,)),
            pltpu.SemaphoreType.DMA((2,)),
        ],
        compiler_params=pltpu.CompilerParams(
            use_tc_tiling_on_sc=False, needs_layout_passes=False
        ),
    )
    def gather_scale(table_hbm, idx_hbm, out_hbm, givec, rows, tbuf, gsem, osem, isem):
        w = lax.axis_index("subcore") * info.num_cores + lax.axis_index("core")
        iota16 = lax.iota(jnp.int32, 16)

        def load_gidx_sync(s, k):
            pltpu.sync_copy(
                idx_hbm.at[pl.ds(s * batch + w * _BBLK, _BBLK)], givec.at[k]
            )

        def start_gidx(s, k):
            pltpu.async_copy(
                idx_hbm.at[pl.ds(s * batch + w * _BBLK, _BBLK)],
                givec.at[k],
                isem.at[k],
            )

        def wait_gidx(k):
            pltpu.make_async_copy(
                idx_hbm.at[pl.ds(0, _BBLK)], givec.at[k], isem.at[k]
            ).wait()

        def start_gather(k):
            pltpu.async_copy(
                table_hbm.at[givec.at[k]], rows.at[k], gsem.at[k]
            )

        def wait_gather(k):
            pltpu.make_async_copy(
                table_hbm.at[pl.ds(0, _BBLK)], rows.at[k], gsem.at[k]
            ).wait()

        def transpose_scale(k):
            src = rows.at[k]
            dst = tbuf.at[k]

            def r_body(r0h, carry):
                riota = iota16 + r0h * _LANES
                for e0 in range(0, _EMBED, _LANES):
                    for d in range(_LANES):
                        ci = e0 + ((iota16 + d) & (_LANES - 1))
                        vals = plsc.load_gather(src, [riota, ci])
                        plsc.store_scatter(dst, [ci, riota], vals * _SCALE)
                return carry

            lax.fori_loop(0, _BBLK // _LANES, r_body, 0)

        def start_out(s, k):
            for eh in range(_EH):
                pltpu.async_copy(
                    tbuf.at[k].at[pl.ds(eh * 8, 8)],
                    out_hbm.at[(s * _EH + eh) * n_workers + w],
                    osem.at[k],
                )

        def wait_out(k):
            for eh in range(_EH):
                pltpu.make_async_copy(
                    tbuf.at[k].at[pl.ds(0, 8)], out_hbm.at[0], osem.at[k]
                ).wait()

        load_gidx_sync(0, 0)
        load_gidx_sync(1, 1)
        start_gather(0)

        def turn(j, carry):
            for k in range(2):
                s = j * 2 + k
                k2 = (k + 1) % 2

                @pl.when(s + 1 < seq)
                def _prefetch(s=s, k2=k2):
                    @pl.when(s >= 1)
                    def _idx_ready(k2=k2):
                        wait_gidx(k2)

                    start_gather(k2)

                wait_gather(k)

                @pl.when(s + 2 < seq)
                def _prefetch_idx(s=s, k=k):
                    start_gidx(s + 2, k)

                @pl.when(s >= 2)
                def _free_tbuf(k=k):
                    wait_out(k)

                transpose_scale(k)
                start_out(s, k)
            return carry

        lax.fori_loop(0, seq // 2, turn, 0)

        for k in range(2):
            wait_out(k)

    out3 = gather_scale(table, idx_t)
    t5 = out3.reshape(seq, _EH, batch // _BBLK, 8, _BBLK)
    return t5.transpose(2, 4, 0, 1, 3).reshape(batch, seq, _EMBED)
